# Initial kernel scaffold; baseline (speedup 1.0000x reference)
import jax
import jax.numpy as jnp
from jax.experimental import pallas as pl


def kernel(voxel_count_gt, pixel_pred, confidence_pred, offset_pred, view_index, velocity_pred, fixedMem, fixedMem_float):
    raise NotImplementedError("write your pallas kernel here")


# speedup vs baseline: 74.9190x; 74.9190x over previous
import functools

import jax
import jax.numpy as jnp
from jax import lax
from jax.experimental import pallas as pl
from jax.experimental.pallas import tpu as pltpu
from jax.experimental.pallas import tpu_sc as plsc

_B, _ROWS, _COLS = 4, 512, 512
_NV = 5
_EXT0, _EXT1 = -51.2, -51.2
_GRID_R = 102.4 / _ROWS
_GRID_C = 102.4 / _COLS
_THRESH = 0.05

_NC, _NS, _L = 2, 16, 16
_NW = _NC * _NS
_ROWS_PER_W = (_B * _ROWS) // _NW
_RC = 8
_NCHUNK = _ROWS_PER_W // _RC
_WPB = _ROWS // _ROWS_PER_W


def _rsqrt(x):
    i = lax.bitcast_convert_type(x, jnp.int32)
    i = jnp.int32(0x5F3759DF) - lax.shift_right_logical(i, 1)
    y = lax.bitcast_convert_type(i, jnp.float32)
    for _ in range(3):
        y = y * (1.5 - 0.5 * x * y * y)
    return y


def _decode_body(voxel, pixel, conf, off, view, vel, out,
                 voxel_b, pixel_b, conf_b, off0_b, off1_b, vel0_b, vel1_b,
                 view_b, out_b, ccol_b):
    wid = lax.axis_index("s") * _NC + lax.axis_index("c")
    b = wid // _WPB
    r_base = (wid % _WPB) * _ROWS_PER_W

    iota = lax.iota(jnp.int32, _L)
    iota5 = iota * _NV
    iota4 = iota * 4
    iota_f = iota.astype(jnp.float32)

    def ccol_body(g, c):
        cb = g * _L
        ccol_b[pl.ds(cb, _L)] = _EXT1 + (cb + iota_f + 0.5) * _GRID_C
        return c
    lax.fori_loop(0, _COLS // _L, ccol_body, 0)

    def chunk_body(ci, carry):
        r0 = r_base + ci * _RC
        pltpu.sync_copy(voxel.at[b, pl.ds(r0 * _COLS, _RC * _COLS)], voxel_b)
        pltpu.sync_copy(pixel.at[b, pl.ds(r0 * _COLS, _RC * _COLS)], pixel_b)
        pltpu.sync_copy(conf.at[b, pl.ds(r0 * _COLS, _RC * _COLS)], conf_b)
        pltpu.sync_copy(off.at[b, 0, pl.ds(r0 * _COLS, _RC * _COLS)], off0_b)
        pltpu.sync_copy(off.at[b, 1, pl.ds(r0 * _COLS, _RC * _COLS)], off1_b)
        pltpu.sync_copy(vel.at[b, 0, pl.ds(r0 * _COLS, _RC * _COLS)], vel0_b)
        pltpu.sync_copy(vel.at[b, 1, pl.ds(r0 * _COLS, _RC * _COLS)], vel1_b)
        pltpu.sync_copy(view.at[b, pl.ds(r0 * _COLS * _NV, _RC * _COLS * _NV)],
                        view_b)

        for row in range(_RC):
            r_glob = (r0 + row).astype(jnp.float32)
            cr_base = _EXT0 + (r_glob + 0.5) * _GRID_R
            rbase = row * _COLS
            vbase = row * _COLS * _NV
            obase = row * _COLS * 4

            def group_body(g, carry2):
                cbase = g * _L
                lin = rbase + cbase
                s = jnp.zeros((_L,), jnp.float32)
                for v in range(_NV):
                    idx_v = plsc.load_gather(
                        view_b, [(vbase + cbase * _NV + v) + iota5])
                    s = s + plsc.load_gather(conf_b, [rbase + idx_v])
                conf_lin = conf_b[pl.ds(lin, _L)]
                conf_final = 0.5 * (conf_lin + s * (1.0 / _NV))

                center_r = cr_base + off0_b[pl.ds(lin, _L)] * _GRID_R
                center_c = (ccol_b[pl.ds(cbase, _L)]
                            + off1_b[pl.ds(lin, _L)] * _GRID_C)

                vx = vel0_b[pl.ds(lin, _L)]
                vy = vel1_b[pl.ds(lin, _L)]
                s2 = vx * vx + vy * vy + 1e-12
                speed = s2 * _rsqrt(s2)

                mask = ((pixel_b[pl.ds(lin, _L)] > _THRESH)
                        & (voxel_b[pl.ds(lin, _L)] > 0))
                neg = jnp.full((_L,), -0.1, jnp.float32)
                out_idx = (obase + cbase * 4) + iota4
                for ch, val in enumerate((center_r, center_c, conf_final,
                                          speed)):
                    plsc.store_scatter(out_b, [out_idx + ch],
                                       jnp.where(mask, val, neg))
                return carry2

            lax.fori_loop(0, _COLS // _L, group_body, 0, unroll=2)

        pltpu.sync_copy(out_b, out.at[b, pl.ds(r0 * _COLS * 4,
                                               _RC * _COLS * 4)])
        return carry

    lax.fori_loop(0, _NCHUNK, chunk_body, 0)


@jax.jit
def _sc_decode(voxel, pixel, conf, off, view, vel):
    mesh = plsc.VectorSubcoreMesh(core_axis_name="c", subcore_axis_name="s")
    f = pl.kernel(
        _decode_body,
        out_type=jax.ShapeDtypeStruct((_B, _ROWS * _COLS * 4), jnp.float32),
        mesh=mesh,
        compiler_params=pltpu.CompilerParams(needs_layout_passes=False),
        scratch_types=[
            pltpu.VMEM((_RC * _COLS,), jnp.int32),
            pltpu.VMEM((_RC * _COLS,), jnp.float32),
            pltpu.VMEM((_RC * _COLS,), jnp.float32),
            pltpu.VMEM((_RC * _COLS,), jnp.float32),
            pltpu.VMEM((_RC * _COLS,), jnp.float32),
            pltpu.VMEM((_RC * _COLS,), jnp.float32),
            pltpu.VMEM((_RC * _COLS,), jnp.float32),
            pltpu.VMEM((_RC * _COLS * _NV,), jnp.int32),
            pltpu.VMEM((_RC * _COLS * 4,), jnp.float32),
            pltpu.VMEM((_COLS,), jnp.float32),
        ],
    )
    return f(voxel.reshape(_B, _ROWS * _COLS),
             pixel.reshape(_B, _ROWS * _COLS),
             conf.reshape(_B, _ROWS * _COLS),
             off.reshape(_B, 2, _ROWS * _COLS),
             view.reshape(_B, _ROWS * _COLS * _NV),
             vel.reshape(_B, 2, _ROWS * _COLS))


def kernel(voxel_count_gt, pixel_pred, confidence_pred, offset_pred,
           view_index, velocity_pred, fixedMem, fixedMem_float):
    out = _sc_decode(voxel_count_gt, pixel_pred, confidence_pred,
                     offset_pred, view_index, velocity_pred)
    return out.reshape(_B, _ROWS, _COLS, 4)


# speedup vs baseline: 95.3333x; 1.2725x over previous
import jax
import jax.numpy as jnp
from jax import lax
from jax.experimental import pallas as pl
from jax.experimental.pallas import tpu as pltpu
from jax.experimental.pallas import tpu_sc as plsc

_B, _ROWS, _COLS = 4, 512, 512
_NV = 5
_EXT0, _EXT1 = -51.2, -51.2
_GRID_R = 102.4 / _ROWS
_GRID_C = 102.4 / _COLS
_THRESH = 0.05

_NC, _NS, _L = 2, 16, 16
_NW = _NC * _NS
_ROWS_PER_W = (_B * _ROWS) // _NW
_RC = 8
_NCHUNK = _ROWS_PER_W // _RC
_WPB = _ROWS // _ROWS_PER_W


def _rsqrt(x):
    i = lax.bitcast_convert_type(x, jnp.int32)
    i = jnp.int32(0x5F3759DF) - lax.shift_right_logical(i, 1)
    y = lax.bitcast_convert_type(i, jnp.float32)
    for _ in range(3):
        y = y * (1.5 - 0.5 * x * y * y)
    return y


def _decode_body(voxel, pixel, conf, off, view, vel, out,
                 voxel_b, pixel_b, conf_b, off0_b, off1_b, vel0_b, vel1_b,
                 view_b, out_b, ccol_b):
    wid = lax.axis_index("s") * _NC + lax.axis_index("c")
    b = wid // _WPB
    r_base = (wid % _WPB) * _ROWS_PER_W

    iota = lax.iota(jnp.int32, _L)
    iota_f = iota.astype(jnp.float32)

    def ccol_body(g, c):
        cb = g * _L
        ccol_b[pl.ds(cb, _L)] = _EXT1 + (cb + iota_f + 0.5) * _GRID_C
        return c
    lax.fori_loop(0, _COLS // _L, ccol_body, 0)

    def chunk_body(ci, carry):
        r0 = r_base + ci * _RC
        pltpu.sync_copy(voxel.at[b, pl.ds(r0, _RC)], voxel_b)
        pltpu.sync_copy(pixel.at[b, pl.ds(r0, _RC)], pixel_b)
        pltpu.sync_copy(conf.at[b, pl.ds(r0, _RC)], conf_b)
        pltpu.sync_copy(off.at[b, 0, pl.ds(r0, _RC)], off0_b)
        pltpu.sync_copy(off.at[b, 1, pl.ds(r0, _RC)], off1_b)
        pltpu.sync_copy(vel.at[b, 0, pl.ds(r0, _RC)], vel0_b)
        pltpu.sync_copy(vel.at[b, 1, pl.ds(r0, _RC)], vel1_b)
        pltpu.sync_copy(view.at[b, pl.ds(r0, _RC)], view_b)

        for row in range(_RC):
            r_glob = (r0 + row).astype(jnp.float32)
            cr_base = _EXT0 + (r_glob + 0.5) * _GRID_R
            row_splat = jnp.full((_L,), row, jnp.int32)

            def group_body(g, carry2):
                cbase = g * _L
                c_vec = cbase + iota
                s = jnp.zeros((_L,), jnp.float32)
                for v in range(_NV):
                    idx_v = plsc.load_gather(
                        view_b,
                        [row_splat, c_vec, jnp.full((_L,), v, jnp.int32)])
                    s = s + plsc.load_gather(conf_b, [row_splat, idx_v])
                conf_lin = conf_b[row, pl.ds(cbase, _L)]
                conf_final = 0.5 * (conf_lin + s * (1.0 / _NV))

                center_r = cr_base + off0_b[row, pl.ds(cbase, _L)] * _GRID_R
                center_c = (ccol_b[pl.ds(cbase, _L)]
                            + off1_b[row, pl.ds(cbase, _L)] * _GRID_C)

                vx = vel0_b[row, pl.ds(cbase, _L)]
                vy = vel1_b[row, pl.ds(cbase, _L)]
                s2 = vx * vx + vy * vy + 1e-12
                speed = s2 * _rsqrt(s2)

                mask = ((pixel_b[row, pl.ds(cbase, _L)] > _THRESH)
                        & (voxel_b[row, pl.ds(cbase, _L)] > 0))
                neg = jnp.full((_L,), -0.1, jnp.float32)
                for ch, val in enumerate((center_r, center_c, conf_final,
                                          speed)):
                    plsc.store_scatter(
                        out_b,
                        [row_splat, c_vec, jnp.full((_L,), ch, jnp.int32)],
                        jnp.where(mask, val, neg))
                return carry2

            lax.fori_loop(0, _COLS // _L, group_body, 0, unroll=2)

        pltpu.sync_copy(out_b, out.at[b, pl.ds(r0, _RC)])
        return carry

    lax.fori_loop(0, _NCHUNK, chunk_body, 0)


@jax.jit
def _sc_decode(voxel, pixel, conf, off, view, vel):
    mesh = plsc.VectorSubcoreMesh(core_axis_name="c", subcore_axis_name="s")
    f = pl.kernel(
        _decode_body,
        out_type=jax.ShapeDtypeStruct((_B, _ROWS, _COLS, 4), jnp.float32),
        mesh=mesh,
        compiler_params=pltpu.CompilerParams(
            needs_layout_passes=False, use_tc_tiling_on_sc=False),
        scratch_types=[
            pltpu.VMEM((_RC, _COLS), jnp.int32),
            pltpu.VMEM((_RC, _COLS), jnp.float32),
            pltpu.VMEM((_RC, _COLS), jnp.float32),
            pltpu.VMEM((_RC, _COLS), jnp.float32),
            pltpu.VMEM((_RC, _COLS), jnp.float32),
            pltpu.VMEM((_RC, _COLS), jnp.float32),
            pltpu.VMEM((_RC, _COLS), jnp.float32),
            pltpu.VMEM((_RC, _COLS, _NV), jnp.int32),
            pltpu.VMEM((_RC, _COLS, 4), jnp.float32),
            pltpu.VMEM((_COLS,), jnp.float32),
        ],
    )
    return f(voxel, pixel, conf, off, view, vel)


def kernel(voxel_count_gt, pixel_pred, confidence_pred, offset_pred,
           view_index, velocity_pred, fixedMem, fixedMem_float):
    return _sc_decode(voxel_count_gt, pixel_pred, confidence_pred,
                      offset_pred, view_index, velocity_pred)


# speedup vs baseline: 189.9045x; 1.9920x over previous
import jax
import jax.numpy as jnp
from jax import lax
from jax.experimental import pallas as pl
from jax.experimental.pallas import tpu as pltpu
from jax.experimental.pallas import tpu_sc as plsc

_B, _ROWS, _COLS = 4, 512, 512
_NV = 5
_EXT0, _EXT1 = -51.2, -51.2
_GRID_R = 102.4 / _ROWS
_GRID_C = 102.4 / _COLS
_THRESH = 0.05

_NC, _NS, _L = 2, 16, 16
_NW = _NC * _NS
_ROWS_PER_W = (_B * _ROWS) // _NW
_RC = 8
_NCHUNK = _ROWS_PER_W // _RC
_WPB = _ROWS // _ROWS_PER_W


def _rsqrt(x):
    i = lax.bitcast_convert_type(x, jnp.int32)
    i = jnp.int32(0x5F3759DF) - lax.shift_right_logical(i, 1)
    y = lax.bitcast_convert_type(i, jnp.float32)
    for _ in range(3):
        y = y * (1.5 - 0.5 * x * y * y)
    return y


def _decode_body(voxel, pixel, conf, off, view0, view1, view2, view3, view4,
                 vel, out,
                 voxel_b, pixel_b, conf_b, off0_b, off1_b, vel0_b, vel1_b,
                 view_b, out_b, ccol_b):
    views = (view0, view1, view2, view3, view4)
    wid = lax.axis_index("s") * _NC + lax.axis_index("c")
    b = wid // _WPB
    r_base = (wid % _WPB) * _ROWS_PER_W

    iota = lax.iota(jnp.int32, _L)
    iota_f = iota.astype(jnp.float32)

    def ccol_body(g, c):
        cb = g * _L
        ccol_b[pl.ds(cb, _L)] = _EXT1 + (cb + iota_f + 0.5) * _GRID_C
        return c
    lax.fori_loop(0, _COLS // _L, ccol_body, 0)

    def chunk_body(ci, carry):
        r0 = r_base + ci * _RC
        pltpu.sync_copy(voxel.at[b, pl.ds(r0, _RC)], voxel_b)
        pltpu.sync_copy(pixel.at[b, pl.ds(r0, _RC)], pixel_b)
        pltpu.sync_copy(conf.at[b, pl.ds(r0, _RC)], conf_b)
        pltpu.sync_copy(off.at[b, 0, pl.ds(r0, _RC)], off0_b)
        pltpu.sync_copy(off.at[b, 1, pl.ds(r0, _RC)], off1_b)
        pltpu.sync_copy(vel.at[b, 0, pl.ds(r0, _RC)], vel0_b)
        pltpu.sync_copy(vel.at[b, 1, pl.ds(r0, _RC)], vel1_b)
        for v in range(_NV):
            pltpu.sync_copy(views[v].at[b, pl.ds(r0, _RC)], view_b.at[v])

        for row in range(_RC):
            r_glob = (r0 + row).astype(jnp.float32)
            cr_base = _EXT0 + (r_glob + 0.5) * _GRID_R
            row_splat = jnp.full((_L,), row, jnp.int32)

            def group_body(g, carry2):
                cbase = g * _L
                c_vec = cbase + iota
                s = jnp.zeros((_L,), jnp.float32)
                for v in range(_NV):
                    idx_v = view_b[v, row, pl.ds(cbase, _L)]
                    s = s + plsc.load_gather(conf_b, [row_splat, idx_v])
                conf_lin = conf_b[row, pl.ds(cbase, _L)]
                conf_final = 0.5 * (conf_lin + s * (1.0 / _NV))

                center_r = cr_base + off0_b[row, pl.ds(cbase, _L)] * _GRID_R
                center_c = (ccol_b[pl.ds(cbase, _L)]
                            + off1_b[row, pl.ds(cbase, _L)] * _GRID_C)

                vx = vel0_b[row, pl.ds(cbase, _L)]
                vy = vel1_b[row, pl.ds(cbase, _L)]
                s2 = vx * vx + vy * vy + 1e-12
                speed = s2 * _rsqrt(s2)

                mask = ((pixel_b[row, pl.ds(cbase, _L)] > _THRESH)
                        & (voxel_b[row, pl.ds(cbase, _L)] > 0))
                neg = jnp.full((_L,), -0.1, jnp.float32)
                for ch, val in enumerate((center_r, center_c, conf_final,
                                          speed)):
                    plsc.store_scatter(
                        out_b,
                        [row_splat, c_vec, jnp.full((_L,), ch, jnp.int32)],
                        jnp.where(mask, val, neg))
                return carry2

            lax.fori_loop(0, _COLS // _L, group_body, 0, unroll=2)

        pltpu.sync_copy(out_b, out.at[b, pl.ds(r0, _RC)])
        return carry

    lax.fori_loop(0, _NCHUNK, chunk_body, 0)


@jax.jit
def _sc_decode(voxel, pixel, conf, off, view, vel):
    mesh = plsc.VectorSubcoreMesh(core_axis_name="c", subcore_axis_name="s")
    f = pl.kernel(
        _decode_body,
        out_type=jax.ShapeDtypeStruct((_B, _ROWS, _COLS, 4), jnp.float32),
        mesh=mesh,
        compiler_params=pltpu.CompilerParams(
            needs_layout_passes=False, use_tc_tiling_on_sc=False),
        scratch_types=[
            pltpu.VMEM((_RC, _COLS), jnp.int32),
            pltpu.VMEM((_RC, _COLS), jnp.float32),
            pltpu.VMEM((_RC, _COLS), jnp.float32),
            pltpu.VMEM((_RC, _COLS), jnp.float32),
            pltpu.VMEM((_RC, _COLS), jnp.float32),
            pltpu.VMEM((_RC, _COLS), jnp.float32),
            pltpu.VMEM((_RC, _COLS), jnp.float32),
            pltpu.VMEM((_NV, _RC, _COLS), jnp.int32),
            pltpu.VMEM((_RC, _COLS, 4), jnp.float32),
            pltpu.VMEM((_COLS,), jnp.float32),
        ],
    )
    return f(voxel, pixel, conf, off,
             view[..., 0], view[..., 1], view[..., 2], view[..., 3],
             view[..., 4], vel)


def kernel(voxel_count_gt, pixel_pred, confidence_pred, offset_pred,
           view_index, velocity_pred, fixedMem, fixedMem_float):
    return _sc_decode(voxel_count_gt, pixel_pred, confidence_pred,
                      offset_pred, view_index, velocity_pred)


# speedup vs baseline: 619.0447x; 3.2598x over previous
import jax
import jax.numpy as jnp
from jax import lax
from jax.experimental import pallas as pl
from jax.experimental.pallas import tpu as pltpu
from jax.experimental.pallas import tpu_sc as plsc

_B, _ROWS, _COLS = 4, 512, 512
_NV = 5
_EXT0, _EXT1 = -51.2, -51.2
_GRID_R = 102.4 / _ROWS
_GRID_C = 102.4 / _COLS
_THRESH = 0.05

_NC, _NS, _L = 2, 16, 16
_NW = _NC * _NS
_ROWS_PER_W = (_B * _ROWS) // _NW
_RC = 8
_NCHUNK = _ROWS_PER_W // _RC
_WPB = _ROWS // _ROWS_PER_W


def _rsqrt(x):
    i = lax.bitcast_convert_type(x, jnp.int32)
    i = jnp.int32(0x5F3759DF) - lax.shift_right_logical(i, 1)
    y = lax.bitcast_convert_type(i, jnp.float32)
    for _ in range(3):
        y = y * (1.5 - 0.5 * x * y * y)
    return y


def _decode_body(voxel, pixel, conf, off, view0, view1, view2, view3, view4,
                 vel, out,
                 voxel_b, pixel_b, conf_b, off0_b, off1_b, vel0_b, vel1_b,
                 view_b, out_b, ccol_b):
    views = (view0, view1, view2, view3, view4)
    wid = lax.axis_index("s") * _NC + lax.axis_index("c")
    b = wid // _WPB
    r_base = (wid % _WPB) * _ROWS_PER_W

    iota = lax.iota(jnp.int32, _L)
    iota_f = iota.astype(jnp.float32)

    def ccol_body(g, c):
        cb = g * _L
        ccol_b[pl.ds(cb, _L)] = _EXT1 + (cb + iota_f + 0.5) * _GRID_C
        return c
    lax.fori_loop(0, _COLS // _L, ccol_body, 0)

    def chunk_body(ci, carry):
        r0 = r_base + ci * _RC
        pltpu.sync_copy(voxel.at[b, pl.ds(r0, _RC)], voxel_b)
        pltpu.sync_copy(pixel.at[b, pl.ds(r0, _RC)], pixel_b)
        pltpu.sync_copy(conf.at[b, pl.ds(r0, _RC)], conf_b)
        pltpu.sync_copy(off.at[b, 0, pl.ds(r0, _RC)], off0_b)
        pltpu.sync_copy(off.at[b, 1, pl.ds(r0, _RC)], off1_b)
        pltpu.sync_copy(vel.at[b, 0, pl.ds(r0, _RC)], vel0_b)
        pltpu.sync_copy(vel.at[b, 1, pl.ds(r0, _RC)], vel1_b)
        for v in range(_NV):
            pltpu.sync_copy(views[v].at[b, pl.ds(r0, _RC)], view_b.at[v])

        for row in range(_RC):
            r_glob = (r0 + row).astype(jnp.float32)
            cr_base = _EXT0 + (r_glob + 0.5) * _GRID_R
            row_splat = jnp.full((_L,), row, jnp.int32)

            def group_body(g, carry2):
                cbase = g * _L
                c_vec = cbase + iota
                s = jnp.zeros((_L,), jnp.float32)
                for v in range(_NV):
                    idx_v = view_b[v, row, pl.ds(cbase, _L)]
                    s = s + plsc.load_gather(conf_b, [row_splat, idx_v])
                conf_lin = conf_b[row, pl.ds(cbase, _L)]
                conf_final = 0.5 * (conf_lin + s * (1.0 / _NV))

                center_r = cr_base + off0_b[row, pl.ds(cbase, _L)] * _GRID_R
                center_c = (ccol_b[pl.ds(cbase, _L)]
                            + off1_b[row, pl.ds(cbase, _L)] * _GRID_C)

                vx = vel0_b[row, pl.ds(cbase, _L)]
                vy = vel1_b[row, pl.ds(cbase, _L)]
                s2 = vx * vx + vy * vy + 1e-12
                speed = s2 * _rsqrt(s2)

                mask = ((pixel_b[row, pl.ds(cbase, _L)] > _THRESH)
                        & (voxel_b[row, pl.ds(cbase, _L)] > 0))
                neg = jnp.full((_L,), -0.1, jnp.float32)
                for ch, val in enumerate((center_r, center_c, conf_final,
                                          speed)):
                    out_b[ch, row, pl.ds(cbase, _L)] = jnp.where(mask, val,
                                                                 neg)
                return carry2

            lax.fori_loop(0, _COLS // _L, group_body, 0, unroll=2)

        for ch in range(4):
            pltpu.sync_copy(out_b.at[ch], out.at[ch, b, pl.ds(r0, _RC)])
        return carry

    lax.fori_loop(0, _NCHUNK, chunk_body, 0)


@jax.jit
def _sc_decode(voxel, pixel, conf, off, view, vel):
    mesh = plsc.VectorSubcoreMesh(core_axis_name="c", subcore_axis_name="s")
    f = pl.kernel(
        _decode_body,
        out_type=jax.ShapeDtypeStruct((4, _B, _ROWS, _COLS), jnp.float32),
        mesh=mesh,
        compiler_params=pltpu.CompilerParams(
            needs_layout_passes=False, use_tc_tiling_on_sc=False),
        scratch_types=[
            pltpu.VMEM((_RC, _COLS), jnp.int32),
            pltpu.VMEM((_RC, _COLS), jnp.float32),
            pltpu.VMEM((_RC, _COLS), jnp.float32),
            pltpu.VMEM((_RC, _COLS), jnp.float32),
            pltpu.VMEM((_RC, _COLS), jnp.float32),
            pltpu.VMEM((_RC, _COLS), jnp.float32),
            pltpu.VMEM((_RC, _COLS), jnp.float32),
            pltpu.VMEM((_NV, _RC, _COLS), jnp.int32),
            pltpu.VMEM((4, _RC, _COLS), jnp.float32),
            pltpu.VMEM((_COLS,), jnp.float32),
        ],
    )
    planar = f(voxel, pixel, conf, off,
               view[..., 0], view[..., 1], view[..., 2], view[..., 3],
               view[..., 4], vel)
    return jnp.transpose(planar, (1, 2, 3, 0))


def kernel(voxel_count_gt, pixel_pred, confidence_pred, offset_pred,
           view_index, velocity_pred, fixedMem, fixedMem_float):
    return _sc_decode(voxel_count_gt, pixel_pred, confidence_pred,
                      offset_pred, view_index, velocity_pred)
